# baseline (speedup 1.0000x reference)
import functools

import jax
import jax.numpy as jnp
from jax import lax
from jax.experimental import pallas as pl
from jax.experimental.pallas import tpu as pltpu
from jax.experimental.pallas import tpu_sc as plsc

_E = 100
_N = 16384
_D = 32
_NW = 32
_SPW = _N // _NW
_S = 128
_NC = _SPW // _S
_NG = _S // 16
_NJ = 28
_NSLOT = 4

_mesh = plsc.VectorSubcoreMesh(core_axis_name="c", subcore_axis_name="s")


def _offsets(j):
    jc = jnp.where(j >= 27, 13, j)
    dr = jc // 9 - 1
    dg = (jc // 3) % 3 - 1
    db = jc % 3 - 1
    return dr, dg, db


@functools.partial(
    pl.kernel,
    mesh=_mesh,
    out_type=jax.ShapeDtypeStruct((_NW, 16), jnp.float32),
    compiler_params=pltpu.CompilerParams(needs_layout_passes=False),
    scratch_types=[
        pltpu.VMEM((_SPW,), jnp.int32),
        pltpu.VMEM((_SPW,), jnp.int32),
        pltpu.VMEM((_SPW,), jnp.int32),
        pltpu.VMEM((_S,), jnp.int32),
        pltpu.VMEM((_S,), jnp.int32),
        pltpu.VMEM((_S, 128), jnp.float32),
        pltpu.VMEM((_S,), jnp.int32),
        pltpu.VMEM((_S,), jnp.int32),
        pltpu.VMEM((_S,), jnp.int32),
        pltpu.VMEM((_S,), jnp.int32),
        pltpu.VMEM((_S,), jnp.int32),
        pltpu.VMEM((_S,), jnp.int32),
        pltpu.VMEM((_S,), jnp.int32),
        pltpu.VMEM((_S,), jnp.int32),
        pltpu.VMEM((_S, 128), jnp.float32),
        pltpu.VMEM((_S, 128), jnp.float32),
        pltpu.VMEM((_S, 128), jnp.float32),
        pltpu.VMEM((_S, 128), jnp.float32),
        pltpu.VMEM((16,), jnp.float32),
        pltpu.SemaphoreType.DMA,
        pltpu.SemaphoreType.DMA,
        pltpu.SemaphoreType.DMA,
        pltpu.SemaphoreType.DMA,
        pltpu.SemaphoreType.DMA,
    ],
)
def _sc_loss(emb128, r_hbm, g_hbm, b_hbm, out,
             r_v, g_v, b_v, ci, cvo, crow,
             i0, i1, i2, i3, v0, v1, v2, v3, b0, b1, b2, b3,
             part, semc, s0, s1, s2, s3):
    idx_bufs = (i0, i1, i2, i3)
    vo_bufs = (v0, v1, v2, v3)
    row_bufs = (b0, b1, b2, b3)
    sems = (s0, s1, s2, s3)

    wid = lax.axis_index("s") * 2 + lax.axis_index("c")
    base = wid * _SPW
    pltpu.sync_copy(r_hbm.at[pl.ds(base, _SPW)], r_v)
    pltpu.sync_copy(g_hbm.at[pl.ds(base, _SPW)], g_v)
    pltpu.sync_copy(b_hbm.at[pl.ds(base, _SPW)], b_v)

    def fill_idx(j, c, idst, vdst):
        dr, dg, db = _offsets(j)

        def body(i, carry):
            src = pl.ds(c * _S + i * 16, 16)
            sl = pl.ds(i * 16, 16)
            rr = jnp.clip(r_v[src] + dr, 0, _E - 1)
            gg = jnp.clip(g_v[src] + dg, 0, _E - 1)
            bb = jnp.clip(b_v[src] + db, 0, _E - 1)
            v = rr + gg * _E + bb * (_E * _E)
            idst[sl] = lax.shift_right_logical(v, 2)
            vdst[sl] = lax.shift_left(jnp.bitwise_and(v, 3), 5)
            return carry

        lax.fori_loop(0, _NG, body, 0, unroll=True)

    def accum(acc, rows, vo):
        def body(i, a):
            s16 = lax.iota(jnp.int32, 16) + i * 16
            vov = vo[pl.ds(i * 16, 16)]
            cov = cvo[pl.ds(i * 16, 16)]
            for j in range(_D):
                g = plsc.load_gather(rows, [s16, vov + j])
                cc = plsc.load_gather(crow, [s16, cov + j])
                d = cc - g
                a = a + d * d
            return a

        return lax.fori_loop(0, _NG, body, acc)

    def chunk(c, acc):
        fill_idx(13, c, ci, cvo)
        ccopy = pltpu.async_copy(emb128.at[ci], crow, semc)
        for jj in range(_NSLOT):
            fill_idx(jj, c, idx_bufs[jj], vo_bufs[jj])
            pltpu.async_copy(emb128.at[idx_bufs[jj]], row_bufs[jj], sems[jj])
        ccopy.wait()

        def outer(t, a):
            for jj in range(_NSLOT):
                j = t * _NSLOT + jj
                pltpu.make_async_copy(
                    emb128.at[idx_bufs[jj]], row_bufs[jj], sems[jj]).wait()
                a = accum(a, row_bufs[jj], vo_bufs[jj])
                fill_idx(j + _NSLOT, c, idx_bufs[jj], vo_bufs[jj])
                pltpu.async_copy(
                    emb128.at[idx_bufs[jj]], row_bufs[jj], sems[jj])
            return a

        acc = lax.fori_loop(0, _NJ // _NSLOT - 1, outer, acc)
        for jj in range(_NSLOT):
            pltpu.make_async_copy(
                emb128.at[idx_bufs[jj]], row_bufs[jj], sems[jj]).wait()
            acc = accum(acc, row_bufs[jj], vo_bufs[jj])
        return acc

    acc = lax.fori_loop(0, _NC, chunk, jnp.zeros((16,), jnp.float32))

    part[...] = acc
    pltpu.sync_copy(part, out.at[wid])


def kernel(embeds):
    k_rgb = jax.random.fold_in(jax.random.key(0), 1)
    rgb = jax.random.randint(k_rgb, (_N, 3), 0, _E, dtype=jnp.int32)
    emb128 = jnp.reshape(embeds, (_E ** 3 // 4, 4 * _D))
    parts = _sc_loss(emb128, rgb[:, 0], rgb[:, 1], rgb[:, 2])
    return jnp.sqrt(jnp.sum(parts))


# speedup vs baseline: 1.4228x; 1.4228x over previous
import functools

import jax
import jax.numpy as jnp
from jax import lax
from jax.experimental import pallas as pl
from jax.experimental.pallas import tpu as pltpu
from jax.experimental.pallas import tpu_sc as plsc

_E = 100
_N = 16384
_D = 32
_NW = 32
_SPW = _N // _NW
_S = 128
_NC = _SPW // _S
_NG = _S // 16
_NJ = 28
_NSLOT = 4

_mesh = plsc.VectorSubcoreMesh(core_axis_name="c", subcore_axis_name="s")


def _offsets(j):
    jc = jnp.where(j >= 27, 13, j)
    dr = jc // 9 - 1
    dg = (jc // 3) % 3 - 1
    db = jc % 3 - 1
    return dr, dg, db


@functools.partial(
    pl.kernel,
    mesh=_mesh,
    out_type=jax.ShapeDtypeStruct((_NW, 16), jnp.float32),
    compiler_params=pltpu.CompilerParams(
        use_tc_tiling_on_sc=True, needs_layout_passes=False),
    scratch_types=[
        pltpu.VMEM((_SPW,), jnp.int32),
        pltpu.VMEM((_SPW,), jnp.int32),
        pltpu.VMEM((_SPW,), jnp.int32),
        pltpu.VMEM((_S,), jnp.int32),
        pltpu.VMEM((_S,), jnp.int32),
        pltpu.VMEM((_S, 128), jnp.float32),
        pltpu.VMEM((_S,), jnp.int32),
        pltpu.VMEM((_S,), jnp.int32),
        pltpu.VMEM((_S,), jnp.int32),
        pltpu.VMEM((_S,), jnp.int32),
        pltpu.VMEM((_S,), jnp.int32),
        pltpu.VMEM((_S,), jnp.int32),
        pltpu.VMEM((_S,), jnp.int32),
        pltpu.VMEM((_S,), jnp.int32),
        pltpu.VMEM((_S, 128), jnp.float32),
        pltpu.VMEM((_S, 128), jnp.float32),
        pltpu.VMEM((_S, 128), jnp.float32),
        pltpu.VMEM((_S, 128), jnp.float32),
        pltpu.VMEM((16,), jnp.float32),
        pltpu.SemaphoreType.DMA,
        pltpu.SemaphoreType.DMA,
        pltpu.SemaphoreType.DMA,
        pltpu.SemaphoreType.DMA,
        pltpu.SemaphoreType.DMA,
    ],
)
def _sc_loss(emb128, r_hbm, g_hbm, b_hbm, out,
             r_v, g_v, b_v, ci, cvo, crow,
             i0, i1, i2, i3, v0, v1, v2, v3, b0, b1, b2, b3,
             part, semc, s0, s1, s2, s3):
    idx_bufs = (i0, i1, i2, i3)
    vo_bufs = (v0, v1, v2, v3)
    row_bufs = (b0, b1, b2, b3)
    sems = (s0, s1, s2, s3)

    wid = lax.axis_index("s") * 2 + lax.axis_index("c")
    base = wid * _SPW
    pltpu.sync_copy(r_hbm.at[pl.ds(base, _SPW)], r_v)
    pltpu.sync_copy(g_hbm.at[pl.ds(base, _SPW)], g_v)
    pltpu.sync_copy(b_hbm.at[pl.ds(base, _SPW)], b_v)

    def fill_idx(j, c, idst, vdst):
        dr, dg, db = _offsets(j)

        def body(i, carry):
            src = pl.ds(c * _S + i * 16, 16)
            sl = pl.ds(i * 16, 16)
            rr = jnp.clip(r_v[src] + dr, 0, _E - 1)
            gg = jnp.clip(g_v[src] + dg, 0, _E - 1)
            bb = jnp.clip(b_v[src] + db, 0, _E - 1)
            v = rr + gg * _E + bb * (_E * _E)
            idst[sl] = lax.shift_right_logical(v, 2)
            vdst[sl] = lax.shift_left(jnp.bitwise_and(v, 3), 5)
            return carry

        lax.fori_loop(0, _NG, body, 0, unroll=True)

    def accum(acc, rows, vo):
        def body(i, a):
            lane = lax.iota(jnp.int32, 16)
            s16 = lane + i * 16
            vov = vo[pl.ds(i * 16, 16)]
            cov = cvo[pl.ds(i * 16, 16)]
            for j in range(_D):
                jj = jnp.bitwise_and(lane + j, _D - 1)
                g = plsc.load_gather(rows, [s16, vov + jj])
                cc = plsc.load_gather(crow, [s16, cov + jj])
                d = cc - g
                a = a + d * d
            return a

        return lax.fori_loop(0, _NG, body, acc)

    def chunk(c, acc):
        fill_idx(13, c, ci, cvo)
        ccopy = pltpu.async_copy(emb128.at[ci], crow, semc)
        for jj in range(_NSLOT):
            fill_idx(jj, c, idx_bufs[jj], vo_bufs[jj])
            pltpu.async_copy(emb128.at[idx_bufs[jj]], row_bufs[jj], sems[jj])
        ccopy.wait()

        def outer(t, a):
            for jj in range(_NSLOT):
                j = t * _NSLOT + jj
                pltpu.make_async_copy(
                    emb128.at[idx_bufs[jj]], row_bufs[jj], sems[jj]).wait()
                a = accum(a, row_bufs[jj], vo_bufs[jj])
                fill_idx(j + _NSLOT, c, idx_bufs[jj], vo_bufs[jj])
                pltpu.async_copy(
                    emb128.at[idx_bufs[jj]], row_bufs[jj], sems[jj])
            return a

        acc = lax.fori_loop(0, _NJ // _NSLOT - 1, outer, acc)
        for jj in range(_NSLOT):
            pltpu.make_async_copy(
                emb128.at[idx_bufs[jj]], row_bufs[jj], sems[jj]).wait()
            acc = accum(acc, row_bufs[jj], vo_bufs[jj])
        return acc

    acc = lax.fori_loop(0, _NC, chunk, jnp.zeros((16,), jnp.float32))

    part[...] = acc
    pltpu.sync_copy(part, out.at[wid])


def kernel(embeds):
    k_rgb = jax.random.fold_in(jax.random.key(0), 1)
    rgb = jax.random.randint(k_rgb, (_N, 3), 0, _E, dtype=jnp.int32)
    emb128 = jnp.reshape(embeds, (_E ** 3 // 4, 4 * _D))
    parts = _sc_loss(emb128, rgb[:, 0], rgb[:, 1], rgb[:, 2])
    return jnp.sqrt(jnp.sum(parts))


# speedup vs baseline: 3.1796x; 2.2348x over previous
import functools

import jax
import jax.numpy as jnp
from jax import lax
from jax.experimental import pallas as pl
from jax.experimental.pallas import tpu as pltpu
from jax.experimental.pallas import tpu_sc as plsc

_E = 100
_N = 16384
_D = 32
_NW = 32
_SPW = _N // _NW
_NVEC = _SPW // 16
_NJ = 28
_NSLOT = 4

_mesh = plsc.VectorSubcoreMesh(core_axis_name="c", subcore_axis_name="s")

_VB = 7812
_TAIL = _E ** 3 - _VB * 128
_BPW = _VB // _NW
_XTRA = _VB - _BPW * _NW


@functools.partial(
    pl.kernel,
    mesh=_mesh,
    out_type=jax.ShapeDtypeStruct((_E ** 3 * _D,), jnp.float32),
    compiler_params=pltpu.CompilerParams(
        use_tc_tiling_on_sc=True, needs_layout_passes=False),
    scratch_types=[
        pltpu.VMEM((_D, 128), jnp.float32),
        pltpu.VMEM((_D, 128), jnp.float32),
        pltpu.VMEM((128 * _D,), jnp.float32),
        pltpu.VMEM((128 * _D,), jnp.float32),
        pltpu.SemaphoreType.DMA,
        pltpu.SemaphoreType.DMA,
        pltpu.SemaphoreType.DMA,
        pltpu.SemaphoreType.DMA,
    ],
)
def _sc_transpose(emb_t, tail_lin, out, in0, in1, ob0, ob1, si0, si1, so0, so1):
    ins = (in0, in1)
    obs = (ob0, ob1)
    sis = (si0, si1)
    sos = (so0, so1)

    wid = lax.axis_index("s") * 2 + lax.axis_index("c")
    nblk = jnp.where(wid < _XTRA, _BPW + 1, _BPW)
    start = wid * _BPW + jnp.minimum(wid, _XTRA)

    def in_src(t):
        return emb_t.at[pl.ds(0, _D), pl.ds((start + t) * 128, 128)]

    def out_dst(t):
        return out.at[pl.ds((start + t) * 128 * _D, 128 * _D)]

    lane = lax.iota(jnp.int32, 16)

    def transpose_chunk(src, dst):
        def dbody(d, carry):
            ddv = jnp.bitwise_and(lane + d, _D - 1)
            for i in range(8):
                vrel = lane + i * 16
                val = plsc.load_gather(src, [ddv, vrel])
                plsc.store_scatter(dst, [vrel * _D + ddv], val)
            return carry

        lax.fori_loop(0, _D, dbody, 0)

    for t in range(2):
        pltpu.async_copy(in_src(t), ins[t], sis[t])

    def body(t, carry):
        slot = jnp.bitwise_and(t, 1)
        for s in range(2):

            @pl.when(slot == s)
            def _():
                pltpu.make_async_copy(in_src(t), ins[s], sis[s]).wait()

                @pl.when(t >= 2)
                def _():
                    pltpu.make_async_copy(obs[s], out_dst(t - 2), sos[s]).wait()

                transpose_chunk(ins[s], obs[s])
                pltpu.async_copy(obs[s], out_dst(t), sos[s])

                @pl.when(t + 2 < nblk)
                def _():
                    pltpu.async_copy(in_src(t + 2), ins[s], sis[s])

        return carry

    lax.fori_loop(0, nblk, body, 0)
    for s in range(2):
        pltpu.make_async_copy(obs[s], out_dst(0), sos[s]).wait()

    @pl.when(wid == _NW - 1)
    def _():
        pltpu.sync_copy(tail_lin, ob0.at[pl.ds(0, _TAIL * _D)])
        pltpu.sync_copy(ob0.at[pl.ds(0, _TAIL * _D)],
                        out.at[pl.ds(_VB * 128 * _D, _TAIL * _D)])


def _offsets(j):
    jc = jnp.where(j >= 27, 13, j)
    dr = jc // 9 - 1
    dg = (jc // 3) % 3 - 1
    db = jc % 3 - 1
    return dr, dg, db


@functools.partial(
    pl.kernel,
    mesh=_mesh,
    out_type=jax.ShapeDtypeStruct((_NW, 16), jnp.float32),
    compiler_params=pltpu.CompilerParams(use_tc_tiling_on_sc=False),
    scratch_types=[
        pltpu.VMEM((_SPW,), jnp.int32),
        pltpu.VMEM((_SPW,), jnp.int32),
        pltpu.VMEM((_SPW,), jnp.int32),
        pltpu.VMEM((_SPW,), jnp.int32),
        pltpu.VMEM((_SPW, _D), jnp.float32),
        pltpu.VMEM((_SPW,), jnp.int32),
        pltpu.VMEM((_SPW,), jnp.int32),
        pltpu.VMEM((_SPW,), jnp.int32),
        pltpu.VMEM((_SPW,), jnp.int32),
        pltpu.VMEM((_SPW, _D), jnp.float32),
        pltpu.VMEM((_SPW, _D), jnp.float32),
        pltpu.VMEM((_SPW, _D), jnp.float32),
        pltpu.VMEM((_SPW, _D), jnp.float32),
        pltpu.VMEM((16,), jnp.float32),
        pltpu.SemaphoreType.DMA,
        pltpu.SemaphoreType.DMA,
        pltpu.SemaphoreType.DMA,
        pltpu.SemaphoreType.DMA,
        pltpu.SemaphoreType.DMA,
    ],
)
def _sc_loss(embeds, r_hbm, g_hbm, b_hbm, out,
             r_v, g_v, b_v, ci, crow,
             i0, i1, i2, i3, b0, b1, b2, b3,
             part, semc, s0, s1, s2, s3):
    idx_bufs = (i0, i1, i2, i3)
    row_bufs = (b0, b1, b2, b3)
    sems = (s0, s1, s2, s3)

    wid = lax.axis_index("s") * 2 + lax.axis_index("c")
    base = wid * _SPW
    pltpu.sync_copy(r_hbm.at[pl.ds(base, _SPW)], r_v)
    pltpu.sync_copy(g_hbm.at[pl.ds(base, _SPW)], g_v)
    pltpu.sync_copy(b_hbm.at[pl.ds(base, _SPW)], b_v)

    def fill_idx(j, dst):
        dr, dg, db = _offsets(j)

        def body(i, carry):
            sl = pl.ds(i * 16, 16)
            rr = jnp.clip(r_v[sl] + dr, 0, _E - 1)
            gg = jnp.clip(g_v[sl] + dg, 0, _E - 1)
            bb = jnp.clip(b_v[sl] + db, 0, _E - 1)
            dst[sl] = rr + gg * _E + bb * (_E * _E)
            return carry

        lax.fori_loop(0, _NVEC, body, 0, unroll=8)

    def accum(acc, rows):
        def body(s, a):
            for h in range(2):
                sl = pl.ds(h * 16, 16)
                d = crow[s, sl] - rows[s, sl]
                a = a + d * d
            return a

        return lax.fori_loop(0, _SPW, body, acc, unroll=8)

    fill_idx(13, ci)
    ccopy = pltpu.async_copy(embeds.at[ci], crow, semc)
    for jj in range(_NSLOT):
        fill_idx(jj, idx_bufs[jj])
        pltpu.async_copy(embeds.at[idx_bufs[jj]], row_bufs[jj], sems[jj])
    ccopy.wait()

    def outer(t, acc):
        for jj in range(_NSLOT):
            j = t * _NSLOT + jj
            pltpu.make_async_copy(
                embeds.at[idx_bufs[jj]], row_bufs[jj], sems[jj]).wait()
            acc = accum(acc, row_bufs[jj])
            fill_idx(j + _NSLOT, idx_bufs[jj])
            pltpu.async_copy(embeds.at[idx_bufs[jj]], row_bufs[jj], sems[jj])
        return acc

    acc = lax.fori_loop(0, _NJ // _NSLOT - 1, outer,
                        jnp.zeros((16,), jnp.float32))
    for jj in range(_NSLOT):
        pltpu.make_async_copy(
            embeds.at[idx_bufs[jj]], row_bufs[jj], sems[jj]).wait()
        acc = accum(acc, row_bufs[jj])

    part[...] = acc
    pltpu.sync_copy(part, out.at[wid])


def kernel(embeds):
    k_rgb = jax.random.fold_in(jax.random.key(0), 1)
    rgb = jax.random.randint(k_rgb, (_N, 3), 0, _E, dtype=jnp.int32)
    tail_lin = jnp.reshape(embeds[_VB * 128:, :], (_TAIL * _D,))
    lin = _sc_transpose(embeds.T, tail_lin)
    table = jnp.reshape(lin, (_E ** 3, _D))
    parts = _sc_loss(table, rgb[:, 0], rgb[:, 1], rgb[:, 2])
    return jnp.sqrt(jnp.sum(parts))
